# baseline (speedup 1.0000x reference)
import functools

import jax
import jax.numpy as jnp
from jax import lax
from jax.experimental import pallas as pl
from jax.experimental.pallas import tpu as pltpu
from jax.experimental.pallas import tpu_sc as plsc

B = 16384
D = 64

_NC = 2
_NS = 16
_NW = _NC * _NS
_BPW = B // _NW
_L = 16


def _sc_gather_body(uid_hbm, mid_hbm, ue_tab, me_tab, ue_out, me_out,
                    idx_u, idx_m, sem_u, sem_m):
  wid = lax.axis_index("s") * _NC + lax.axis_index("c")
  base = wid * _BPW

  pltpu.sync_copy(uid_hbm.at[pl.ds(base, _BPW)], idx_u)
  pltpu.sync_copy(mid_hbm.at[pl.ds(base, _BPW)], idx_m)

  lanes = jax.lax.broadcasted_iota(jnp.int32, (_L,), 0)
  zeros = jnp.zeros((_L,), jnp.int32)

  def issue(j, _):
    chunk_u = idx_u[pl.ds(j * _L, _L)]
    chunk_m = idx_m[pl.ds(j * _L, _L)]
    for l in range(_L):
      i = j * _L + l
      ru = jnp.sum(jnp.where(lanes == l, chunk_u, zeros))
      rm = jnp.sum(jnp.where(lanes == l, chunk_m, zeros))
      pltpu.async_copy(ue_tab.at[pl.ds(ru, 1), :],
                       ue_out.at[pl.ds(base + i, 1), :], sem_u)
      pltpu.async_copy(me_tab.at[pl.ds(rm, 1), :],
                       me_out.at[pl.ds(base + i, 1), :], sem_m)
    return 0

  lax.fori_loop(0, _BPW // _L, issue, 0, unroll=False)

  pltpu.make_async_copy(ue_tab.at[pl.ds(0, _BPW), :],
                        ue_out.at[pl.ds(base, _BPW), :], sem_u).wait()
  pltpu.make_async_copy(me_tab.at[pl.ds(0, _BPW), :],
                        me_out.at[pl.ds(base, _BPW), :], sem_m).wait()


def _sc_gather(user_ids, movie_ids, user_emb, movie_emb):
  mesh = plsc.VectorSubcoreMesh(core_axis_name="c", subcore_axis_name="s",
                                num_cores=_NC, num_subcores=_NS)
  f = pl.kernel(
      _sc_gather_body,
      out_type=(
          jax.ShapeDtypeStruct((B, D), jnp.float32),
          jax.ShapeDtypeStruct((B, D), jnp.float32),
      ),
      mesh=mesh,
      compiler_params=pltpu.CompilerParams(use_tc_tiling_on_sc=True,
                                           needs_layout_passes=False),
      scratch_types=[
          pltpu.VMEM((_BPW,), jnp.int32),
          pltpu.VMEM((_BPW,), jnp.int32),
          pltpu.SemaphoreType.DMA,
          pltpu.SemaphoreType.DMA,
      ],
  )
  return f(user_ids, movie_ids, user_emb, movie_emb)


def _mlp_body(genres, ue, me, bias, gWT, gb, W0aT, W0bT, W0cT, b0,
              W1T, b1, W2T, b2, Wo, bo, out):
  gf = jnp.maximum(
      jnp.dot(genres[...], gWT[...], preferred_element_type=jnp.float32)
      + gb[...], 0.0)
  h = (jnp.dot(ue[...], W0aT[...], preferred_element_type=jnp.float32)
       + jnp.dot(me[...], W0bT[...], preferred_element_type=jnp.float32)
       + jnp.dot(gf, W0cT[...], preferred_element_type=jnp.float32)
       + b0[...])
  h = jnp.maximum(h, 0.0)
  h = jnp.maximum(
      jnp.dot(h, W1T[...], preferred_element_type=jnp.float32) + b1[...], 0.0)
  h = jnp.maximum(
      jnp.dot(h, W2T[...], preferred_element_type=jnp.float32) + b2[...], 0.0)
  base = jnp.sum(h * Wo[...], axis=1) + bo[0, 0]
  final = base + bias[...]
  out[...] = jax.nn.sigmoid(final) * 5.0


def _mlp(genres, ue, me, bias, gWT, gb, W0aT, W0bT, W0cT, b0, W1T, b1,
         W2T, b2, Wo, bo):
  blk = 2048
  grid = (B // blk,)
  NG = genres.shape[1]

  def rows(i):
    return (i, 0)

  def full(i):
    return (0, 0)

  def vec(i):
    return (i,)

  in_specs = [
      pl.BlockSpec((blk, NG), rows),
      pl.BlockSpec((blk, D), rows),
      pl.BlockSpec((blk, D), rows),
      pl.BlockSpec((blk,), vec),
      pl.BlockSpec(gWT.shape, full),
      pl.BlockSpec(gb.shape, full),
      pl.BlockSpec(W0aT.shape, full),
      pl.BlockSpec(W0bT.shape, full),
      pl.BlockSpec(W0cT.shape, full),
      pl.BlockSpec(b0.shape, full),
      pl.BlockSpec(W1T.shape, full),
      pl.BlockSpec(b1.shape, full),
      pl.BlockSpec(W2T.shape, full),
      pl.BlockSpec(b2.shape, full),
      pl.BlockSpec(Wo.shape, full),
      pl.BlockSpec(bo.shape, full),
  ]
  return pl.pallas_call(
      _mlp_body,
      grid=grid,
      in_specs=in_specs,
      out_specs=pl.BlockSpec((blk,), vec),
      out_shape=jax.ShapeDtypeStruct((B,), jnp.float32),
  )(genres, ue, me, bias, gWT, gb, W0aT, W0bT, W0cT, b0, W1T, b1, W2T,
    b2, Wo, bo)


def kernel(user_ids, movie_ids, genres, user_emb, movie_emb, user_bias,
           movie_bias, genre_W, genre_b, W0, b0, W1, b1, W2, b2, Wo, bo):
  ue, me = _sc_gather(user_ids, movie_ids, user_emb, movie_emb)
  ub = jnp.take(user_bias, user_ids, axis=0)[:, 0]
  mb = jnp.take(movie_bias, movie_ids, axis=0)[:, 0]
  bias = ub + mb
  gWT = genre_W.T
  W0aT = W0[:, :D].T
  W0bT = W0[:, D:2 * D].T
  W0cT = W0[:, 2 * D:].T
  return _mlp(genres, ue, me, bias,
              gWT, genre_b.reshape(1, -1),
              W0aT, W0bT, W0cT, b0.reshape(1, -1),
              W1.T, b1.reshape(1, -1),
              W2.T, b2.reshape(1, -1),
              Wo, bo.reshape(1, 1))


# speedup vs baseline: 1.3795x; 1.3795x over previous
import functools

import jax
import jax.numpy as jnp
from jax import lax
from jax.experimental import pallas as pl
from jax.experimental.pallas import tpu as pltpu
from jax.experimental.pallas import tpu_sc as plsc

B = 16384
D = 64

_NC = 2
_NS = 16
_NW = _NC * _NS
_BPW = B // _NW
_CH = _BPW // 2
_L = 16


def _sc_gather_body(uid_hbm, mid_hbm, ue_tab, me_tab, ue_out, me_out,
                    idx_u, idx_m, div_u, div_m, rows_u, rows_m,
                    sem_u, sem_m):
  wid = lax.axis_index("s") * _NC + lax.axis_index("c")
  base = wid * _BPW

  pltpu.sync_copy(uid_hbm.at[pl.ds(base, _BPW)], idx_u)
  pltpu.sync_copy(mid_hbm.at[pl.ds(base, _BPW)], idx_m)

  def div_body(j, _):
    s = pl.ds(j * _L, _L)
    div_u[s] = lax.shift_right_logical(idx_u[s], 1)
    div_m[s] = lax.shift_right_logical(idx_m[s], 1)
    return 0

  lax.fori_loop(0, _BPW // _L, div_body, 0, unroll=False)

  for p in range(_BPW // _CH):
    s = pl.ds(p * _CH, _CH)
    cu = pltpu.async_copy(ue_tab.at[div_u.at[s]], rows_u, sem_u)
    cm = pltpu.async_copy(me_tab.at[div_m.at[s]], rows_m, sem_m)
    cu.wait()
    pltpu.sync_copy(rows_u, ue_out.at[pl.ds(base + p * _CH, _CH)])
    cm.wait()
    pltpu.sync_copy(rows_m, me_out.at[pl.ds(base + p * _CH, _CH)])


def _sc_gather(user_ids, movie_ids, user_emb, movie_emb):
  mesh = plsc.VectorSubcoreMesh(core_axis_name="c", subcore_axis_name="s",
                                num_cores=_NC, num_subcores=_NS)
  f = pl.kernel(
      _sc_gather_body,
      out_type=(
          jax.ShapeDtypeStruct((B, 2 * D), jnp.float32),
          jax.ShapeDtypeStruct((B, 2 * D), jnp.float32),
      ),
      mesh=mesh,
      compiler_params=pltpu.CompilerParams(use_tc_tiling_on_sc=True,
                                           needs_layout_passes=False),
      scratch_types=[
          pltpu.VMEM((_BPW,), jnp.int32),
          pltpu.VMEM((_BPW,), jnp.int32),
          pltpu.VMEM((_BPW,), jnp.int32),
          pltpu.VMEM((_BPW,), jnp.int32),
          pltpu.VMEM((_CH, 2 * D), jnp.float32),
          pltpu.VMEM((_CH, 2 * D), jnp.float32),
          pltpu.SemaphoreType.DMA,
          pltpu.SemaphoreType.DMA,
      ],
  )
  return f(user_ids, movie_ids,
           user_emb.reshape(-1, 2 * D), movie_emb.reshape(-1, 2 * D))


def _mlp_body(genres, ue2, me2, uids, mids, bias, gWT, gb, W0aT, W0bT,
              W0cT, b0, W1T, b1, W2T, b2, Wo, bo, out):
  pu = lax.bitwise_and(uids[...], 1)[:, None]
  pm = lax.bitwise_and(mids[...], 1)[:, None]
  ue = jnp.where(pu == 1, ue2[..., D:], ue2[..., :D])
  me = jnp.where(pm == 1, me2[..., D:], me2[..., :D])
  h = _tower(genres, ue, me, bias, gWT, gb, W0aT, W0bT, W0cT, b0, W1T,
             b1, W2T, b2, Wo, bo)
  out[...] = h


def _tower(genres, ue, me, bias, gWT, gb, W0aT, W0bT, W0cT, b0, W1T, b1,
           W2T, b2, Wo, bo):
  gf = jnp.maximum(
      jnp.dot(genres[...], gWT[...], preferred_element_type=jnp.float32)
      + gb[...], 0.0)
  h = (jnp.dot(ue[...], W0aT[...], preferred_element_type=jnp.float32)
       + jnp.dot(me[...], W0bT[...], preferred_element_type=jnp.float32)
       + jnp.dot(gf, W0cT[...], preferred_element_type=jnp.float32)
       + b0[...])
  h = jnp.maximum(h, 0.0)
  h = jnp.maximum(
      jnp.dot(h, W1T[...], preferred_element_type=jnp.float32) + b1[...], 0.0)
  h = jnp.maximum(
      jnp.dot(h, W2T[...], preferred_element_type=jnp.float32) + b2[...], 0.0)
  base = jnp.sum(h * Wo[...], axis=1) + bo[0, 0]
  final = base + bias[...]
  return jax.nn.sigmoid(final) * 5.0


def _mlp(genres, ue2, me2, uids, mids, bias, gWT, gb, W0aT, W0bT, W0cT,
         b0, W1T, b1, W2T, b2, Wo, bo):
  blk = 2048
  grid = (B // blk,)
  NG = genres.shape[1]

  def rows(i):
    return (i, 0)

  def full(i):
    return (0, 0)

  def vec(i):
    return (i,)

  in_specs = [
      pl.BlockSpec((blk, NG), rows),
      pl.BlockSpec((blk, 2 * D), rows),
      pl.BlockSpec((blk, 2 * D), rows),
      pl.BlockSpec((blk,), vec),
      pl.BlockSpec((blk,), vec),
      pl.BlockSpec((blk,), vec),
      pl.BlockSpec(gWT.shape, full),
      pl.BlockSpec(gb.shape, full),
      pl.BlockSpec(W0aT.shape, full),
      pl.BlockSpec(W0bT.shape, full),
      pl.BlockSpec(W0cT.shape, full),
      pl.BlockSpec(b0.shape, full),
      pl.BlockSpec(W1T.shape, full),
      pl.BlockSpec(b1.shape, full),
      pl.BlockSpec(W2T.shape, full),
      pl.BlockSpec(b2.shape, full),
      pl.BlockSpec(Wo.shape, full),
      pl.BlockSpec(bo.shape, full),
  ]
  return pl.pallas_call(
      _mlp_body,
      grid=grid,
      in_specs=in_specs,
      out_specs=pl.BlockSpec((blk,), vec),
      out_shape=jax.ShapeDtypeStruct((B,), jnp.float32),
  )(genres, ue2, me2, uids, mids, bias, gWT, gb, W0aT, W0bT, W0cT, b0,
    W1T, b1, W2T, b2, Wo, bo)


def kernel(user_ids, movie_ids, genres, user_emb, movie_emb, user_bias,
           movie_bias, genre_W, genre_b, W0, b0, W1, b1, W2, b2, Wo, bo):
  ue2, me2 = _sc_gather(user_ids, movie_ids, user_emb, movie_emb)
  ub = jnp.take(user_bias, user_ids, axis=0)[:, 0]
  mb = jnp.take(movie_bias, movie_ids, axis=0)[:, 0]
  bias = ub + mb
  gWT = genre_W.T
  W0aT = W0[:, :D].T
  W0bT = W0[:, D:2 * D].T
  W0cT = W0[:, 2 * D:].T
  return _mlp(genres, ue2, me2, user_ids, movie_ids, bias,
              gWT, genre_b.reshape(1, -1),
              W0aT, W0bT, W0cT, b0.reshape(1, -1),
              W1.T, b1.reshape(1, -1),
              W2.T, b2.reshape(1, -1),
              Wo, bo.reshape(1, 1))
